# baseline (speedup 1.0000x reference)
import functools

import jax
import jax.numpy as jnp
from jax import lax
from jax.experimental import pallas as pl
from jax.experimental.pallas import tpu as pltpu
from jax.experimental.pallas import tpu_sc as plsc

B = 16384
ED = 256
HD = 512
CD = 256
K = 8192
BETA = 0.25

BB = 256
KC = 512
NKC = K // KC


def _mm(a, b, dims):
    return lax.dot_general(a, b, (dims, ((), ())),
                           precision=lax.Precision.DEFAULT,
                           preferred_element_type=jnp.float32)


def _enc_body(subj_ref, rel_ref, obj_ref, w1_ref, b1_ref, w2_ref, b2_ref,
              w3_ref, b3_ref, z_ref, pv_ref):
    pv_ref[:, 0:ED] = subj_ref[...]
    pv_ref[:, ED:2 * ED] = rel_ref[...]
    pv_ref[:, 2 * ED:3 * ED] = obj_ref[...]
    h = jnp.maximum(_mm(pv_ref[...], w1_ref[...], ((1,), (0,))) + b1_ref[...], 0.0)
    h = jnp.maximum(_mm(h, w2_ref[...], ((1,), (0,))) + b2_ref[...], 0.0)
    z_ref[...] = _mm(h, w3_ref[...], ((1,), (0,))) + b3_ref[...]


def _encoder(subj, rel, obj, w1, b1, w2, b2, w3, b3):
    full = lambda i: (0, 0)
    return pl.pallas_call(
        _enc_body,
        grid=(B // BB,),
        in_specs=[
            pl.BlockSpec((BB, ED), lambda i: (i, 0)),
            pl.BlockSpec((BB, ED), lambda i: (i, 0)),
            pl.BlockSpec((BB, ED), lambda i: (i, 0)),
            pl.BlockSpec((3 * ED, HD), full),
            pl.BlockSpec((1, HD), full),
            pl.BlockSpec((HD, HD), full),
            pl.BlockSpec((1, HD), full),
            pl.BlockSpec((HD, CD), full),
            pl.BlockSpec((1, CD), full),
        ],
        out_specs=pl.BlockSpec((BB, CD), lambda i: (i, 0)),
        out_shape=jax.ShapeDtypeStruct((B, CD), jnp.float32),
        scratch_shapes=[pltpu.VMEM((BB, 3 * ED), jnp.float32)],
    )(subj, rel, obj, w1, b1, w2, b2, w3, b3)


_GB = (0, 2731, 5462, 8192)


def _vq_body(z_ref, zs_ref, cs_ref, cb_ref, codes_ref):
    z = z_ref[...]
    zs = zs_ref[...]
    gmin = [jnp.full((BB, 1), jnp.inf, jnp.float32) for _ in range(3)]
    gidx = [jnp.zeros((BB, 1), jnp.int32) for _ in range(3)]
    iota = lax.broadcasted_iota(jnp.int32, (BB, KC), 1)
    for c in range(NKC):
        lo = c * KC
        hi = lo + KC
        cb_c = cb_ref[lo:hi, :]
        t = _mm(z, cb_c, ((1,), (1,)))
        d = (zs + cs_ref[:, lo:hi]) - 2.0 * t
        for g in range(3):
            glo, ghi = _GB[g], _GB[g + 1]
            if ghi <= lo or glo >= hi:
                continue
            if glo <= lo and hi <= ghi:
                dg = d
            else:
                mask = (iota >= (glo - lo)) & (iota < (ghi - lo))
                dg = jnp.where(mask, d, jnp.inf)
            m = jnp.min(dg, axis=1, keepdims=True)
            idx = jnp.min(jnp.where(dg == m, iota, KC), axis=1, keepdims=True) + lo
            better = m < gmin[g]
            gidx[g] = jnp.where(better, idx, gidx[g])
            gmin[g] = jnp.where(better, m, gmin[g])
    acc_v = gmin[0].astype(jnp.bfloat16).astype(jnp.float32)
    acc_i = gidx[0]
    for g in (1, 2):
        keep_val = acc_v < gmin[g]
        keep_idx = keep_val | (acc_v == gmin[g])
        acc_i = jnp.where(keep_idx, acc_i, gidx[g])
        acc_v = jnp.where(keep_val, acc_v, gmin[g]).astype(jnp.bfloat16).astype(jnp.float32)
    codes_ref[...] = acc_i


def _vq(z, zs, cs_row, codebook):
    full = lambda i: (0, 0)
    return pl.pallas_call(
        _vq_body,
        grid=(B // BB,),
        in_specs=[
            pl.BlockSpec((BB, CD), lambda i: (i, 0)),
            pl.BlockSpec((BB, 1), lambda i: (i, 0)),
            pl.BlockSpec((1, K), full),
            pl.BlockSpec((K, CD), full),
        ],
        out_specs=pl.BlockSpec((BB, 1), lambda i: (i, 0)),
        out_shape=jax.ShapeDtypeStruct((B, 1), jnp.int32),
    )(z, zs, cs_row, codebook)


_NC = 2
_NS = 16
_NW = _NC * _NS
_BPW = B // _NW
_GCH = 4
_BPC = _BPW // _GCH


@functools.cache
def _sc_gather_fn():
    @functools.partial(
        pl.kernel,
        out_type=jax.ShapeDtypeStruct((B, CD), jnp.float32),
        mesh=plsc.VectorSubcoreMesh(core_axis_name="c", subcore_axis_name="s"),
        scratch_types=[
            pltpu.VMEM((2, _BPC), jnp.int32),
            pltpu.VMEM((2, _BPC, CD), jnp.float32),
            pltpu.SemaphoreType.DMA,
            pltpu.SemaphoreType.DMA,
        ],
    )
    def _sc_gather(cb_hbm, idx_hbm, out_hbm, idx_v, rows_v, gsem, wsem):
        wid = lax.axis_index("s") * _NC + lax.axis_index("c")
        base = wid * _BPW
        gh = [None] * _GCH
        wh = [None] * _GCH
        for ch in range(_GCH):
            b = ch % 2
            if ch >= 2:
                wh[ch - 2].wait()
            pltpu.sync_copy(idx_hbm.at[pl.ds(base + ch * _BPC, _BPC)], idx_v.at[b])
            gh[ch] = pltpu.async_copy(cb_hbm.at[idx_v.at[b]], rows_v.at[b], gsem)
            if ch >= 1:
                gh[ch - 1].wait()
                wh[ch - 1] = pltpu.async_copy(
                    rows_v.at[(ch - 1) % 2],
                    out_hbm.at[pl.ds(base + (ch - 1) * _BPC, _BPC)], wsem)
        gh[_GCH - 1].wait()
        wh[_GCH - 2].wait()
        wh[_GCH - 1] = pltpu.async_copy(
            rows_v.at[(_GCH - 1) % 2],
            out_hbm.at[pl.ds(base + (_GCH - 1) * _BPC, _BPC)], wsem)
        wh[_GCH - 1].wait()

    return _sc_gather


def _dec_body(z_ref, zq_ref, w1_ref, b1_ref, w2_ref, b2_ref, w3_ref, b3_ref,
              subj_ref, rel_ref, obj_ref, loss_ref):
    z = z_ref[...]
    zq = zq_ref[...]
    diff = zq - z
    zq_st = z + diff
    h = jnp.maximum(_mm(zq_st, w1_ref[...], ((1,), (0,))) + b1_ref[...], 0.0)
    h = jnp.maximum(_mm(h, w2_ref[...], ((1,), (0,))) + b2_ref[...], 0.0)
    out = _mm(h, w3_ref[...], ((1,), (0,))) + b3_ref[...]
    subj_ref[...] = out[:, 0:ED]
    rel_ref[...] = out[:, ED:2 * ED]
    obj_ref[...] = out[:, 2 * ED:3 * ED]

    part = jnp.sum(diff * diff)

    @pl.when(pl.program_id(0) == 0)
    def _init():
        loss_ref[...] = jnp.zeros_like(loss_ref)

    loss_ref[...] = loss_ref[...] + part


def _decoder(z, zq, w1, b1, w2, b2, w3, b3):
    full = lambda i: (0, 0)
    return pl.pallas_call(
        _dec_body,
        grid=(B // BB,),
        in_specs=[
            pl.BlockSpec((BB, CD), lambda i: (i, 0)),
            pl.BlockSpec((BB, CD), lambda i: (i, 0)),
            pl.BlockSpec((CD, HD), full),
            pl.BlockSpec((1, HD), full),
            pl.BlockSpec((HD, HD), full),
            pl.BlockSpec((1, HD), full),
            pl.BlockSpec((HD, 3 * ED), full),
            pl.BlockSpec((1, 3 * ED), full),
        ],
        out_specs=[
            pl.BlockSpec((BB, ED), lambda i: (i, 0)),
            pl.BlockSpec((BB, ED), lambda i: (i, 0)),
            pl.BlockSpec((BB, ED), lambda i: (i, 0)),
            pl.BlockSpec((1, 1), full),
        ],
        out_shape=[
            jax.ShapeDtypeStruct((B, ED), jnp.float32),
            jax.ShapeDtypeStruct((B, ED), jnp.float32),
            jax.ShapeDtypeStruct((B, ED), jnp.float32),
            jax.ShapeDtypeStruct((1, 1), jnp.float32),
        ],
    )(z, zq, w1, b1, w2, b2, w3, b3)


def kernel(subj_emb, rel_emb, obj_emb, enc_W1, enc_b1, enc_W2, enc_b2,
           enc_W3, enc_b3, codebook, dec_W1, dec_b1, dec_W2, dec_b2,
           dec_W3, dec_b3):
    z = _encoder(
        subj_emb, rel_emb, obj_emb,
        enc_W1, enc_b1.reshape(1, HD),
        enc_W2, enc_b2.reshape(1, HD),
        enc_W3, enc_b3.reshape(1, CD))
    zs = jnp.sum(z ** 2, axis=1, keepdims=True)
    cs_row = jnp.sum(codebook ** 2, axis=1).reshape(1, K)
    codes2d = _vq(z, zs, cs_row, codebook)
    codes = codes2d.reshape(B)
    zq = _sc_gather_fn()(codebook, codes)
    subj_recon, rel_recon, obj_recon, loss_sum = _decoder(
        z, zq,
        dec_W1, dec_b1.reshape(1, HD),
        dec_W2, dec_b2.reshape(1, HD),
        dec_W3, dec_b3.reshape(1, 3 * ED))
    codebook_loss = (loss_sum[0, 0] / jnp.float32(B * CD)).reshape(())
    commitment_loss = codebook_loss * jnp.float32(BETA)
    total_vq = codebook_loss + commitment_loss
    return (subj_recon, rel_recon, obj_recon, codes,
            codebook_loss, commitment_loss, total_vq)


# speedup vs baseline: 1.1052x; 1.1052x over previous
import functools

import jax
import jax.numpy as jnp
from jax import lax
from jax.experimental import pallas as pl
from jax.experimental.pallas import tpu as pltpu
from jax.experimental.pallas import tpu_sc as plsc

B = 16384
ED = 256
HD = 512
CD = 256
K = 8192
BETA = 0.25

BB = 256
KC = 1024
NKC = K // KC


def _mm(a, b, dims):
    return lax.dot_general(a, b, (dims, ((), ())),
                           precision=lax.Precision.DEFAULT,
                           preferred_element_type=jnp.float32)


def _enc_body(subj_ref, rel_ref, obj_ref, w1_ref, b1_ref, w2_ref, b2_ref,
              w3_ref, b3_ref, z_ref, pv_ref):
    pv_ref[:, 0:ED] = subj_ref[...]
    pv_ref[:, ED:2 * ED] = rel_ref[...]
    pv_ref[:, 2 * ED:3 * ED] = obj_ref[...]
    h = jnp.maximum(_mm(pv_ref[...], w1_ref[...], ((1,), (0,))) + b1_ref[...], 0.0)
    h = jnp.maximum(_mm(h, w2_ref[...], ((1,), (0,))) + b2_ref[...], 0.0)
    z_ref[...] = _mm(h, w3_ref[...], ((1,), (0,))) + b3_ref[...]


def _encoder(subj, rel, obj, w1, b1, w2, b2, w3, b3):
    full = lambda i: (0, 0)
    return pl.pallas_call(
        _enc_body,
        grid=(B // BB,),
        in_specs=[
            pl.BlockSpec((BB, ED), lambda i: (i, 0)),
            pl.BlockSpec((BB, ED), lambda i: (i, 0)),
            pl.BlockSpec((BB, ED), lambda i: (i, 0)),
            pl.BlockSpec((3 * ED, HD), full),
            pl.BlockSpec((1, HD), full),
            pl.BlockSpec((HD, HD), full),
            pl.BlockSpec((1, HD), full),
            pl.BlockSpec((HD, CD), full),
            pl.BlockSpec((1, CD), full),
        ],
        out_specs=pl.BlockSpec((BB, CD), lambda i: (i, 0)),
        out_shape=jax.ShapeDtypeStruct((B, CD), jnp.float32),
        scratch_shapes=[pltpu.VMEM((BB, 3 * ED), jnp.float32)],
    )(subj, rel, obj, w1, b1, w2, b2, w3, b3)


_GB = (0, 2731, 5462, 8192)


def _vq_body(z_ref, zs_ref, cs_ref, cb_ref, codes_ref):
    z = z_ref[...]
    zs = zs_ref[...]
    gmin = [jnp.full((BB, 1), jnp.inf, jnp.float32) for _ in range(3)]
    gidx = [jnp.zeros((BB, 1), jnp.int32) for _ in range(3)]
    iota = lax.broadcasted_iota(jnp.int32, (BB, KC), 1)
    for c in range(NKC):
        lo = c * KC
        hi = lo + KC
        cb_c = cb_ref[lo:hi, :]
        t = _mm(z, cb_c, ((1,), (1,)))
        d = (zs + cs_ref[:, lo:hi]) - 2.0 * t
        for g in range(3):
            glo, ghi = _GB[g], _GB[g + 1]
            if ghi <= lo or glo >= hi:
                continue
            if glo <= lo and hi <= ghi:
                dg = d
            else:
                mask = (iota >= (glo - lo)) & (iota < (ghi - lo))
                dg = jnp.where(mask, d, jnp.inf)
            m = jnp.min(dg, axis=1, keepdims=True)
            idx = jnp.min(jnp.where(dg == m, iota, KC), axis=1, keepdims=True) + lo
            better = m < gmin[g]
            gidx[g] = jnp.where(better, idx, gidx[g])
            gmin[g] = jnp.where(better, m, gmin[g])
    acc_v = gmin[0].astype(jnp.bfloat16).astype(jnp.float32)
    acc_i = gidx[0]
    for g in (1, 2):
        keep_val = acc_v < gmin[g]
        keep_idx = keep_val | (acc_v == gmin[g])
        acc_i = jnp.where(keep_idx, acc_i, gidx[g])
        acc_v = jnp.where(keep_val, acc_v, gmin[g]).astype(jnp.bfloat16).astype(jnp.float32)
    codes_ref[...] = acc_i


def _vq(z, zs, cs_row, codebook):
    full = lambda i: (0, 0)
    return pl.pallas_call(
        _vq_body,
        grid=(B // BB,),
        in_specs=[
            pl.BlockSpec((BB, CD), lambda i: (i, 0)),
            pl.BlockSpec((BB, 1), lambda i: (i, 0)),
            pl.BlockSpec((1, K), full),
            pl.BlockSpec((K, CD), full),
        ],
        out_specs=pl.BlockSpec((BB, 1), lambda i: (i, 0)),
        out_shape=jax.ShapeDtypeStruct((B, 1), jnp.int32),
    )(z, zs, cs_row, codebook)


_NC = 2
_NS = 16
_NW = _NC * _NS
_BPW = B // _NW
_GCH = 4
_BPC = _BPW // _GCH


@functools.cache
def _sc_gather_fn():
    @functools.partial(
        pl.kernel,
        out_type=jax.ShapeDtypeStruct((B, CD), jnp.float32),
        mesh=plsc.VectorSubcoreMesh(core_axis_name="c", subcore_axis_name="s"),
        scratch_types=[
            pltpu.VMEM((2, _BPC), jnp.int32),
            pltpu.VMEM((2, _BPC, CD), jnp.float32),
            pltpu.SemaphoreType.DMA,
            pltpu.SemaphoreType.DMA,
        ],
    )
    def _sc_gather(cb_hbm, idx_hbm, out_hbm, idx_v, rows_v, gsem, wsem):
        wid = lax.axis_index("s") * _NC + lax.axis_index("c")
        base = wid * _BPW
        gh = [None] * _GCH
        wh = [None] * _GCH
        for ch in range(_GCH):
            b = ch % 2
            if ch >= 2:
                wh[ch - 2].wait()
            pltpu.sync_copy(idx_hbm.at[pl.ds(base + ch * _BPC, _BPC)], idx_v.at[b])
            gh[ch] = pltpu.async_copy(cb_hbm.at[idx_v.at[b]], rows_v.at[b], gsem)
            if ch >= 1:
                gh[ch - 1].wait()
                wh[ch - 1] = pltpu.async_copy(
                    rows_v.at[(ch - 1) % 2],
                    out_hbm.at[pl.ds(base + (ch - 1) * _BPC, _BPC)], wsem)
        gh[_GCH - 1].wait()
        wh[_GCH - 2].wait()
        wh[_GCH - 1] = pltpu.async_copy(
            rows_v.at[(_GCH - 1) % 2],
            out_hbm.at[pl.ds(base + (_GCH - 1) * _BPC, _BPC)], wsem)
        wh[_GCH - 1].wait()

    return _sc_gather


def _dec_body(z_ref, zq_ref, w1_ref, b1_ref, w2_ref, b2_ref, w3_ref, b3_ref,
              subj_ref, rel_ref, obj_ref, loss_ref):
    z = z_ref[...]
    zq = zq_ref[...]
    diff = zq - z
    zq_st = z + diff
    h = jnp.maximum(_mm(zq_st, w1_ref[...], ((1,), (0,))) + b1_ref[...], 0.0)
    h = jnp.maximum(_mm(h, w2_ref[...], ((1,), (0,))) + b2_ref[...], 0.0)
    out = _mm(h, w3_ref[...], ((1,), (0,))) + b3_ref[...]
    subj_ref[...] = out[:, 0:ED]
    rel_ref[...] = out[:, ED:2 * ED]
    obj_ref[...] = out[:, 2 * ED:3 * ED]

    part = jnp.sum(diff * diff)

    @pl.when(pl.program_id(0) == 0)
    def _init():
        loss_ref[...] = jnp.zeros_like(loss_ref)

    loss_ref[...] = loss_ref[...] + part


def _decoder(z, zq, w1, b1, w2, b2, w3, b3):
    full = lambda i: (0, 0)
    return pl.pallas_call(
        _dec_body,
        grid=(B // BB,),
        in_specs=[
            pl.BlockSpec((BB, CD), lambda i: (i, 0)),
            pl.BlockSpec((BB, CD), lambda i: (i, 0)),
            pl.BlockSpec((CD, HD), full),
            pl.BlockSpec((1, HD), full),
            pl.BlockSpec((HD, HD), full),
            pl.BlockSpec((1, HD), full),
            pl.BlockSpec((HD, 3 * ED), full),
            pl.BlockSpec((1, 3 * ED), full),
        ],
        out_specs=[
            pl.BlockSpec((BB, ED), lambda i: (i, 0)),
            pl.BlockSpec((BB, ED), lambda i: (i, 0)),
            pl.BlockSpec((BB, ED), lambda i: (i, 0)),
            pl.BlockSpec((1, 1), full),
        ],
        out_shape=[
            jax.ShapeDtypeStruct((B, ED), jnp.float32),
            jax.ShapeDtypeStruct((B, ED), jnp.float32),
            jax.ShapeDtypeStruct((B, ED), jnp.float32),
            jax.ShapeDtypeStruct((1, 1), jnp.float32),
        ],
    )(z, zq, w1, b1, w2, b2, w3, b3)


def kernel(subj_emb, rel_emb, obj_emb, enc_W1, enc_b1, enc_W2, enc_b2,
           enc_W3, enc_b3, codebook, dec_W1, dec_b1, dec_W2, dec_b2,
           dec_W3, dec_b3):
    z = _encoder(
        subj_emb, rel_emb, obj_emb,
        enc_W1, enc_b1.reshape(1, HD),
        enc_W2, enc_b2.reshape(1, HD),
        enc_W3, enc_b3.reshape(1, CD))
    zs = jnp.sum(z ** 2, axis=1, keepdims=True)
    cs_row = jnp.sum(codebook ** 2, axis=1).reshape(1, K)
    codes2d = _vq(z, zs, cs_row, codebook)
    codes = codes2d.reshape(B)
    zq = _sc_gather_fn()(codebook, codes)
    subj_recon, rel_recon, obj_recon, loss_sum = _decoder(
        z, zq,
        dec_W1, dec_b1.reshape(1, HD),
        dec_W2, dec_b2.reshape(1, HD),
        dec_W3, dec_b3.reshape(1, 3 * ED))
    codebook_loss = (loss_sum[0, 0] / jnp.float32(B * CD)).reshape(())
    commitment_loss = codebook_loss * jnp.float32(BETA)
    total_vq = codebook_loss + commitment_loss
    return (subj_recon, rel_recon, obj_recon, codes,
            codebook_loss, commitment_loss, total_vq)
